# baseline (speedup 1.0000x reference)
import functools

import jax
import jax.numpy as jnp
from jax import lax
from jax.experimental import pallas as pl
from jax.experimental.pallas import tpu as pltpu
from jax.experimental.pallas import tpu_sc as plsc

NBUF = 2


def _make_sc_kernel(batch, seq_len, num_workers, scale):
    seq_per_w = batch // num_workers
    half = seq_len // 2
    split = min(128, seq_len)
    rest = seq_len - split

    def body(x_hbm, table_hbm, pe_hbm, out_hbm, idx_v, pe_v, rows_v, res_v, gsem, ssem):
        cid = lax.axis_index("c")
        sid = lax.axis_index("s")
        wid = sid * 2 + cid
        sbase = wid * seq_per_w

        pltpu.sync_copy(x_hbm.at[pl.ds(sbase, seq_per_w)], idx_v)
        pltpu.sync_copy(pe_hbm, pe_v)

        def start_gather(t, slot):
            pltpu.async_copy(
                table_hbm.at[idx_v.at[t, pl.ds(0, split)]],
                rows_v.at[slot, pl.ds(0, split)],
                gsem.at[slot],
            )
            if rest:
                pltpu.async_copy(
                    table_hbm.at[idx_v.at[t, pl.ds(split, rest)]],
                    rows_v.at[slot, pl.ds(split, rest)],
                    gsem.at[slot],
                )

        def wait_gather(t, slot):
            pltpu.make_async_copy(
                table_hbm.at[idx_v.at[t, pl.ds(0, split)]],
                rows_v.at[slot, pl.ds(0, split)],
                gsem.at[slot],
            ).wait()
            if rest:
                pltpu.make_async_copy(
                    table_hbm.at[idx_v.at[t, pl.ds(split, rest)]],
                    rows_v.at[slot, pl.ds(split, rest)],
                    gsem.at[slot],
                ).wait()

        def wait_scatter(slot):
            pltpu.make_async_copy(
                res_v.at[slot], out_hbm.at[0], ssem.at[slot]
            ).wait()

        def compute(slot):
            def rbody(r2, carry):
                for g in range(8):
                    pos = g // 4
                    sl = pl.ds(g * 16, 16)
                    rsl = pl.ds((g % 4) * 16, 16)
                    res_v[slot, r2, sl] = (
                        rows_v[slot, 2 * r2 + pos, rsl] * scale + pe_v[r2, sl]
                    )
                return carry

            lax.fori_loop(0, half, rbody, 0, unroll=2)

        start_gather(0, 0)

        def outer(o, carry):
            for b in range(NBUF):
                t = o * NBUF + b

                @pl.when(t + 1 < seq_per_w)
                def _():
                    start_gather(t + 1, (b + 1) % NBUF)

                wait_gather(t, b)

                @pl.when(t >= NBUF)
                def _():
                    wait_scatter(b)

                compute(b)
                pltpu.async_copy(res_v.at[b], out_hbm.at[sbase + t], ssem.at[b])
            return carry

        lax.fori_loop(0, seq_per_w // NBUF, outer, 0)

        for b in range(NBUF):
            wait_scatter(b)

    mesh = plsc.VectorSubcoreMesh(core_axis_name="c", subcore_axis_name="s")
    return pl.kernel(
        body,
        out_type=jax.ShapeDtypeStruct((batch, half, 128), jnp.float32),
        mesh=mesh,
        compiler_params=pltpu.CompilerParams(use_tc_tiling_on_sc=False),
        scratch_types=[
            pltpu.VMEM((seq_per_w, seq_len), jnp.int32),
            pltpu.VMEM((half, 128), jnp.float32),
            pltpu.VMEM((NBUF, seq_len, 64), jnp.float32),
            pltpu.VMEM((NBUF, half, 128), jnp.float32),
            pltpu.SemaphoreType.DMA((NBUF,)),
            pltpu.SemaphoreType.DMA((NBUF,)),
        ],
    )


@jax.jit
def kernel(x_vals, seq_lengths, table, pe):
    batch, seq_len = x_vals.shape
    emb_dim = table.shape[1]
    scale = table.shape[1] ** 1 / 2
    pe2 = pe.reshape(pe.shape[-2], pe.shape[-1])[:seq_len].reshape(seq_len // 2, 128)
    k = _make_sc_kernel(batch, seq_len, 32, scale)
    out = k(x_vals.astype(jnp.int32), table, pe2)
    return out.reshape(batch, seq_len, emb_dim), seq_lengths


# speedup vs baseline: 1.1779x; 1.1779x over previous
import functools

import jax
import jax.numpy as jnp
from jax import lax
from jax.experimental import pallas as pl
from jax.experimental.pallas import tpu as pltpu
from jax.experimental.pallas import tpu_sc as plsc

NGBUF = 4
NRBUF = 2
LA = 2


def _make_sc_kernel(batch, seq_len, num_workers, scale):
    seq_per_w = batch // num_workers
    half = seq_len // 2
    split = min(128, seq_len)
    rest = seq_len - split

    def body(x_hbm, table_hbm, pe_hbm, out_hbm, idx_v, pe_v, rows_v, res_v, gsem, ssem):
        cid = lax.axis_index("c")
        sid = lax.axis_index("s")
        wid = sid * 2 + cid
        sbase = wid * seq_per_w

        pltpu.sync_copy(x_hbm.at[pl.ds(sbase, seq_per_w)], idx_v)
        pltpu.sync_copy(pe_hbm, pe_v)

        def start_gather(t, slot):
            pltpu.async_copy(
                table_hbm.at[idx_v.at[t, pl.ds(0, split)]],
                rows_v.at[slot, pl.ds(0, split)],
                gsem.at[slot],
            )
            if rest:
                pltpu.async_copy(
                    table_hbm.at[idx_v.at[t, pl.ds(split, rest)]],
                    rows_v.at[slot, pl.ds(split, rest)],
                    gsem.at[slot],
                )

        def wait_gather(t, slot):
            pltpu.make_async_copy(
                table_hbm.at[idx_v.at[t, pl.ds(0, split)]],
                rows_v.at[slot, pl.ds(0, split)],
                gsem.at[slot],
            ).wait()
            if rest:
                pltpu.make_async_copy(
                    table_hbm.at[idx_v.at[t, pl.ds(split, rest)]],
                    rows_v.at[slot, pl.ds(split, rest)],
                    gsem.at[slot],
                ).wait()

        def wait_scatter(slot):
            pltpu.make_async_copy(
                res_v.at[slot], out_hbm.at[0], ssem.at[slot]
            ).wait()

        def compute(gslot, rslot):
            def rbody(r2, carry):
                for g in range(8):
                    pos = g // 4
                    sl = pl.ds(g * 16, 16)
                    rsl = pl.ds((g % 4) * 16, 16)
                    res_v[rslot, r2, sl] = (
                        rows_v[gslot, 2 * r2 + pos, rsl] * scale + pe_v[r2, sl]
                    )
                return carry

            lax.fori_loop(0, half, rbody, 0, unroll=2)

        for t0 in range(LA):
            start_gather(t0, t0)

        def outer(o, carry):
            for b in range(NGBUF):
                t = o * NGBUF + b
                rb = b % NRBUF

                @pl.when(t + LA < seq_per_w)
                def _():
                    start_gather(t + LA, (b + LA) % NGBUF)

                wait_gather(t, b)

                @pl.when(t >= NRBUF)
                def _():
                    wait_scatter(rb)

                compute(b, rb)
                pltpu.async_copy(res_v.at[rb], out_hbm.at[sbase + t], ssem.at[rb])
            return carry

        lax.fori_loop(0, seq_per_w // NGBUF, outer, 0)

        for rb in range(NRBUF):
            wait_scatter(rb)

    mesh = plsc.VectorSubcoreMesh(core_axis_name="c", subcore_axis_name="s")
    return pl.kernel(
        body,
        out_type=jax.ShapeDtypeStruct((batch, half, 128), jnp.float32),
        mesh=mesh,
        compiler_params=pltpu.CompilerParams(use_tc_tiling_on_sc=False),
        scratch_types=[
            pltpu.VMEM((seq_per_w, seq_len), jnp.int32),
            pltpu.VMEM((half, 128), jnp.float32),
            pltpu.VMEM((NGBUF, seq_len, 64), jnp.float32),
            pltpu.VMEM((NRBUF, half, 128), jnp.float32),
            pltpu.SemaphoreType.DMA((NGBUF,)),
            pltpu.SemaphoreType.DMA((NRBUF,)),
        ],
    )


@jax.jit
def kernel(x_vals, seq_lengths, table, pe):
    batch, seq_len = x_vals.shape
    emb_dim = table.shape[1]
    scale = table.shape[1] ** 1 / 2
    pe2 = pe.reshape(pe.shape[-2], pe.shape[-1])[:seq_len].reshape(seq_len // 2, 128)
    k = _make_sc_kernel(batch, seq_len, 32, scale)
    out = k(x_vals.astype(jnp.int32), table, pe2)
    return out.reshape(batch, seq_len, emb_dim), seq_lengths
